# baseline (speedup 1.0000x reference)
import functools

import jax
import jax.numpy as jnp
from jax import lax
from jax.experimental import pallas as pl
from jax.experimental.pallas import tpu as pltpu
from jax.experimental.pallas import tpu_sc as plsc

_N_E = 8192
_E_DIM = 256
_BETA = 0.25
_N_TOK = 18432

_BN = 2048
_BE = 4096
_NJ = _N_E // _BE

_N_A = 12288
_N_B = _N_TOK - _N_A

_LOSS_SCALE = (1.0 + _BETA) / (_N_TOK * _E_DIM)


def _argmin_body(zn_ref, en_ref, z_ref, e_ref, idx_ref, loss_ref):
    i = pl.program_id(0)
    z2 = -2.0 * z_ref[...]
    zn = zn_ref[...]
    run_min = jnp.full((_BN, 1), jnp.inf, jnp.float32)
    run_idx = jnp.zeros((_BN, 1), jnp.float32)
    lanes_f = lax.broadcasted_iota(jnp.int32, (_BN, 128), 1) \
        .astype(jnp.float32)
    for j in range(_NJ):
        e_blk = e_ref[j * _BE:(j + 1) * _BE, :]
        en_blk = en_ref[:, j * _BE:(j + 1) * _BE]
        t2 = lax.dot_general(z2, e_blk, (((1,), (1,)), ((), ())),
                             preferred_element_type=jnp.float32)
        m = (zn + en_blk[:, :128]) + t2[:, :128]
        gid = jnp.zeros((_BN, 128), jnp.float32)
        for g in range(1, _BE // 128):
            v = (zn + en_blk[:, g * 128:(g + 1) * 128]) \
                + t2[:, g * 128:(g + 1) * 128]
            upd = v < m
            gid = jnp.where(upd, float(g), gid)
            m = jnp.minimum(m, v)
        lmin = jnp.min(m, axis=1, keepdims=True)
        key = jnp.where(m == lmin, gid * 128.0 + lanes_f, float(_N_E))
        lidx = jnp.min(key, axis=1, keepdims=True) + float(j * _BE)
        better = lmin < run_min
        run_idx = jnp.where(better, lidx, run_idx)
        run_min = jnp.where(better, lmin, run_min)
    idx_ref[...] = run_idx.astype(jnp.int32)

    @pl.when(i == 0)
    def _():
        loss_ref[...] = jnp.zeros_like(loss_ref)

    loss_ref[...] = loss_ref[...] + jnp.sum(run_min)


def _make_argmin_call(nrows, row_offset):
    nb = nrows // _BN
    off = row_offset // _BN

    return pl.pallas_call(
        _argmin_body,
        grid=(nb,),
        in_specs=[
            pl.BlockSpec((_BN, 1), lambda i: (i + off, 0)),
            pl.BlockSpec((1, _N_E), lambda i: (0, 0)),
            pl.BlockSpec((_BN, _E_DIM), lambda i: (i + off, 0)),
            pl.BlockSpec((_N_E, _E_DIM), lambda i: (0, 0)),
        ],
        out_specs=[
            pl.BlockSpec((_BN, 1), lambda i: (i, 0)),
            pl.BlockSpec((1, 1), lambda i: (0, 0)),
        ],
        out_shape=[
            jax.ShapeDtypeStruct((nrows, 1), jnp.int32),
            jax.ShapeDtypeStruct((1, 1), jnp.float32),
        ],
    )


_argmin_a = _make_argmin_call(_N_A, 0)
_argmin_b = _make_argmin_call(_N_B, _N_A)


_NC = 2
_NS = 16
_NW = _NC * _NS
_CH = 96


@functools.cache
def _make_gather(nrows):
    bpw = nrows // _NW
    nch = bpw // _CH

    def body(idx_hbm, tab_hbm, out_hbm, idx_v, rows_v, sem):
        wid = lax.axis_index("s") * _NC + lax.axis_index("c")
        base = wid * bpw
        for c in range(nch):
            off = base + c * _CH
            pltpu.sync_copy(idx_hbm.at[pl.ds(off, _CH)], idx_v)
            pltpu.async_copy(tab_hbm.at[idx_v], rows_v, sem).wait()
            pltpu.sync_copy(rows_v, out_hbm.at[pl.ds(off, _CH)])

    return pl.kernel(
        body,
        out_type=jax.ShapeDtypeStruct((nrows, _E_DIM), jnp.float32),
        mesh=plsc.VectorSubcoreMesh(core_axis_name="c", subcore_axis_name="s",
                                    num_cores=_NC, num_subcores=_NS),
        scratch_types=[
            pltpu.VMEM((_CH,), jnp.int32),
            pltpu.VMEM((_CH, _E_DIM), jnp.float32),
            pltpu.SemaphoreType.DMA,
        ],
    )


def kernel(z, embedding_weight):
    zn = jnp.sum(z ** 2, axis=1, keepdims=True)
    en = jnp.sum(embedding_weight ** 2, axis=1).reshape(1, _N_E)
    idx_a, loss_a = _argmin_a(zn, en, z, embedding_weight)
    z_q_a = _make_gather(_N_A)(idx_a.reshape(_N_A), embedding_weight)
    idx_b, loss_b = _argmin_b(zn, en, z, embedding_weight)
    z_q_b = _make_gather(_N_B)(idx_b.reshape(_N_B), embedding_weight)
    z_q = jnp.concatenate([z_q_a, z_q_b], axis=0)
    loss = (loss_a + loss_b).reshape(()) * _LOSS_SCALE
    return (z_q, loss)


# speedup vs baseline: 1.0912x; 1.0912x over previous
import functools

import jax
import jax.numpy as jnp
from jax import lax
from jax.experimental import pallas as pl
from jax.experimental.pallas import tpu as pltpu
from jax.experimental.pallas import tpu_sc as plsc

_N_E = 8192
_E_DIM = 256
_BETA = 0.25
_N_TOK = 18432

_BN = 2048
_BE = 4096
_NJ = _N_E // _BE
_NB = _N_TOK // _BN

_LOSS_SCALE = (1.0 + _BETA) / (_N_TOK * _E_DIM)


def _argmin_body(zn_ref, en_ref, z_ref, e_ref, idx_ref, loss_ref):
    i = pl.program_id(0)
    z2 = -2.0 * z_ref[...]
    zn = zn_ref[...]
    run_min = jnp.full((_BN, 1), jnp.inf, jnp.float32)
    run_idx = jnp.zeros((_BN, 1), jnp.float32)
    lanes_f = lax.broadcasted_iota(jnp.int32, (_BN, 128), 1) \
        .astype(jnp.float32)
    for j in range(_NJ):
        e_blk = e_ref[j * _BE:(j + 1) * _BE, :]
        en_blk = en_ref[:, j * _BE:(j + 1) * _BE]
        t2 = lax.dot_general(z2, e_blk, (((1,), (1,)), ((), ())),
                             preferred_element_type=jnp.float32)
        m = (zn + en_blk[:, :128]) + t2[:, :128]
        gid = jnp.zeros((_BN, 128), jnp.float32)
        for g in range(1, _BE // 128):
            v = (zn + en_blk[:, g * 128:(g + 1) * 128]) \
                + t2[:, g * 128:(g + 1) * 128]
            upd = v < m
            gid = jnp.where(upd, float(g), gid)
            m = jnp.minimum(m, v)
        lmin = jnp.min(m, axis=1, keepdims=True)
        key = jnp.where(m == lmin, gid * 128.0 + lanes_f, float(_N_E))
        lidx = jnp.min(key, axis=1, keepdims=True) + float(j * _BE)
        better = lmin < run_min
        run_idx = jnp.where(better, lidx, run_idx)
        run_min = jnp.where(better, lmin, run_min)
    idx_ref[...] = run_idx.astype(jnp.int32)

    @pl.when(i == 0)
    def _():
        loss_ref[...] = jnp.zeros_like(loss_ref)

    loss_ref[...] = loss_ref[...] + jnp.sum(run_min)

    @pl.when(i == _NB - 1)
    def _():
        loss_ref[...] = loss_ref[...] * _LOSS_SCALE


_argmin_call = pl.pallas_call(
    _argmin_body,
    grid=(_NB,),
    in_specs=[
        pl.BlockSpec((_BN, 1), lambda i: (i, 0)),
        pl.BlockSpec((1, _N_E), lambda i: (0, 0)),
        pl.BlockSpec((_BN, _E_DIM), lambda i: (i, 0)),
        pl.BlockSpec((_N_E, _E_DIM), lambda i: (0, 0)),
    ],
    out_specs=[
        pl.BlockSpec((_BN, 1), lambda i: (i, 0)),
        pl.BlockSpec((1, 1), lambda i: (0, 0)),
    ],
    out_shape=[
        jax.ShapeDtypeStruct((_N_TOK, 1), jnp.int32),
        jax.ShapeDtypeStruct((1, 1), jnp.float32),
    ],
)


_NC = 2
_NS = 16
_NW = _NC * _NS
_BPW = _N_TOK // _NW
_CH = 96
_NCH = _BPW // _CH


def _gather_body(idx_hbm, tab_hbm, out_hbm, idx_v, rows_v, sem):
    wid = lax.axis_index("s") * _NC + lax.axis_index("c")
    base = wid * _BPW
    for c in range(_NCH):
        off = base + c * _CH
        pltpu.sync_copy(idx_hbm.at[pl.ds(off, _CH)], idx_v)
        pltpu.async_copy(tab_hbm.at[idx_v], rows_v, sem).wait()
        pltpu.sync_copy(rows_v, out_hbm.at[pl.ds(off, _CH)])


@functools.cache
def _get_gather_call():
    return pl.kernel(
        _gather_body,
        out_type=jax.ShapeDtypeStruct((_N_TOK, _E_DIM), jnp.float32),
        mesh=plsc.VectorSubcoreMesh(core_axis_name="c", subcore_axis_name="s",
                                    num_cores=_NC, num_subcores=_NS),
        scratch_types=[
            pltpu.VMEM((_CH,), jnp.int32),
            pltpu.VMEM((_CH, _E_DIM), jnp.float32),
            pltpu.SemaphoreType.DMA,
        ],
    )


def kernel(z, embedding_weight):
    zn = jnp.sum(z ** 2, axis=1, keepdims=True)
    en = jnp.sum(embedding_weight ** 2, axis=1).reshape(1, _N_E)
    idx2, loss = _argmin_call(zn, en, z, embedding_weight)
    z_q = _get_gather_call()(idx2.reshape(_N_TOK), embedding_weight)
    return (z_q, loss.reshape(()))
